# baseline (speedup 1.0000x reference)
import jax
import jax.numpy as jnp
from jax.experimental import pallas as pl
from jax.experimental.pallas import tpu as pltpu

_HIDDEN = 2048
_Q_SIZE = 2048
_KV_SIZE = 512
_OUT_SIZE = _Q_SIZE + 2 * _KV_SIZE
_R = 16
_SCALING = 2.0

_BM = 256
_BNM = 512


def _merge_body(w_ref, b_ref, a_ref, weff_ref):
    ba = jax.lax.dot_general(
        b_ref[...], a_ref[...], (((1,), (0,)), ((), ())),
        preferred_element_type=jnp.float32)
    weff_ref[...] = (
        w_ref[...].astype(jnp.float32) + _SCALING * ba
    ).astype(jnp.bfloat16)


def _matmul_body(x_ref, weff_ref, o_ref):
    o_ref[...] = jax.lax.dot_general(
        x_ref[...].astype(jnp.bfloat16), weff_ref[...],
        (((1,), (1,)), ((), ())),
        preferred_element_type=jnp.float32)


def kernel(x, weight, lora_A, lora_B_q, lora_B_k, lora_B_v):
    orig_shape = x.shape
    x_flat = x.reshape(-1, x.shape[-1])
    m_total = x_flat.shape[0]

    b_exp = jnp.zeros((_OUT_SIZE, 3 * _R), jnp.float32)
    b_exp = b_exp.at[:_Q_SIZE, :_R].set(lora_B_q[0])
    b_exp = b_exp.at[_Q_SIZE:_Q_SIZE + _KV_SIZE, _R:2 * _R].set(lora_B_k[0])
    b_exp = b_exp.at[_Q_SIZE + _KV_SIZE:, 2 * _R:].set(lora_B_v[0])
    a_stack = lora_A[0].reshape(3 * _R, _HIDDEN)

    weff = pl.pallas_call(
        _merge_body,
        grid=(_OUT_SIZE // _BNM,),
        in_specs=[
            pl.BlockSpec((_BNM, _HIDDEN), lambda n: (n, 0)),
            pl.BlockSpec((_BNM, 3 * _R), lambda n: (n, 0)),
            pl.BlockSpec((3 * _R, _HIDDEN), lambda n: (0, 0)),
        ],
        out_specs=pl.BlockSpec((_BNM, _HIDDEN), lambda n: (n, 0)),
        out_shape=jax.ShapeDtypeStruct((_OUT_SIZE, _HIDDEN), jnp.bfloat16),
        compiler_params=pltpu.CompilerParams(
            dimension_semantics=("parallel",)),
    )(weight, b_exp, a_stack)

    out = pl.pallas_call(
        _matmul_body,
        grid=(m_total // _BM,),
        in_specs=[
            pl.BlockSpec((_BM, _HIDDEN), lambda m: (m, 0)),
            pl.BlockSpec((_OUT_SIZE, _HIDDEN), lambda m: (0, 0)),
        ],
        out_specs=pl.BlockSpec((_BM, _OUT_SIZE), lambda m: (m, 0)),
        out_shape=jax.ShapeDtypeStruct((m_total, _OUT_SIZE), jnp.float32),
        compiler_params=pltpu.CompilerParams(
            dimension_semantics=("parallel",)),
    )(x_flat, weff)
    return out.reshape(*orig_shape[:-1], _OUT_SIZE)


# speedup vs baseline: 1.0162x; 1.0162x over previous
import jax
import jax.numpy as jnp
from jax.experimental import pallas as pl
from jax.experimental.pallas import tpu as pltpu

_HIDDEN = 2048
_Q_SIZE = 2048
_KV_SIZE = 512
_OUT_SIZE = _Q_SIZE + 2 * _KV_SIZE
_R = 16
_SCALING = 2.0

_BM = 1024
_BNM = 512


def _merge_body(w_ref, b_ref, a_ref, weff_ref):
    ab = jax.lax.dot_general(
        a_ref[...], b_ref[...], (((0,), (1,)), ((), ())),
        preferred_element_type=jnp.float32)
    wt = jnp.transpose(w_ref[...]).astype(jnp.float32)
    weff_ref[...] = (wt + _SCALING * ab).astype(jnp.bfloat16)


def _matmul_body(x_ref, weff_ref, o_ref):
    o_ref[...] = jax.lax.dot_general(
        x_ref[...].astype(jnp.bfloat16), weff_ref[...],
        (((1,), (0,)), ((), ())),
        preferred_element_type=jnp.float32)


def kernel(x, weight, lora_A, lora_B_q, lora_B_k, lora_B_v):
    orig_shape = x.shape
    x_flat = x.reshape(-1, x.shape[-1])
    m_total = x_flat.shape[0]

    b_exp = jnp.zeros((_OUT_SIZE, 3 * _R), jnp.float32)
    b_exp = b_exp.at[:_Q_SIZE, :_R].set(lora_B_q[0])
    b_exp = b_exp.at[_Q_SIZE:_Q_SIZE + _KV_SIZE, _R:2 * _R].set(lora_B_k[0])
    b_exp = b_exp.at[_Q_SIZE + _KV_SIZE:, 2 * _R:].set(lora_B_v[0])
    a_stack = lora_A[0].reshape(3 * _R, _HIDDEN)

    weff = pl.pallas_call(
        _merge_body,
        grid=(_OUT_SIZE // _BNM,),
        in_specs=[
            pl.BlockSpec((_BNM, _HIDDEN), lambda n: (n, 0)),
            pl.BlockSpec((_BNM, 3 * _R), lambda n: (n, 0)),
            pl.BlockSpec((3 * _R, _HIDDEN), lambda n: (0, 0)),
        ],
        out_specs=pl.BlockSpec((_HIDDEN, _BNM), lambda n: (0, n)),
        out_shape=jax.ShapeDtypeStruct((_HIDDEN, _OUT_SIZE), jnp.bfloat16),
        compiler_params=pltpu.CompilerParams(
            dimension_semantics=("parallel",)),
    )(weight, b_exp, a_stack)

    out = pl.pallas_call(
        _matmul_body,
        grid=(m_total // _BM,),
        in_specs=[
            pl.BlockSpec((_BM, _HIDDEN), lambda m: (m, 0)),
            pl.BlockSpec((_HIDDEN, _OUT_SIZE), lambda m: (0, 0)),
        ],
        out_specs=pl.BlockSpec((_BM, _OUT_SIZE), lambda m: (m, 0)),
        out_shape=jax.ShapeDtypeStruct((m_total, _OUT_SIZE), jnp.float32),
        compiler_params=pltpu.CompilerParams(
            dimension_semantics=("parallel",)),
    )(x_flat, weff)
    return out.reshape(*orig_shape[:-1], _OUT_SIZE)


# speedup vs baseline: 1.0460x; 1.0293x over previous
import jax
import jax.numpy as jnp
from jax.experimental import pallas as pl
from jax.experimental.pallas import tpu as pltpu

_HIDDEN = 2048
_Q_SIZE = 2048
_KV_SIZE = 512
_OUT_SIZE = _Q_SIZE + 2 * _KV_SIZE
_R = 16
_SCALING = 2.0

_BM = 256


def _body(x_ref, w_ref, b_ref, a_ref, o_ref, weff_ref):
    @pl.when(pl.program_id(0) == 0)
    def _merge():
        ba = jax.lax.dot_general(
            b_ref[...], a_ref[...], (((1,), (0,)), ((), ())),
            preferred_element_type=jnp.float32)
        weff_ref[...] = (
            w_ref[...].astype(jnp.float32) + _SCALING * ba
        ).astype(jnp.bfloat16)

    o_ref[...] = jax.lax.dot_general(
        x_ref[...].astype(jnp.bfloat16), weff_ref[...],
        (((1,), (1,)), ((), ())),
        preferred_element_type=jnp.float32)


def kernel(x, weight, lora_A, lora_B_q, lora_B_k, lora_B_v):
    orig_shape = x.shape
    x_flat = x.reshape(-1, x.shape[-1])
    m_total = x_flat.shape[0]

    b_exp = jnp.zeros((_OUT_SIZE, 3 * _R), jnp.float32)
    b_exp = b_exp.at[:_Q_SIZE, :_R].set(lora_B_q[0])
    b_exp = b_exp.at[_Q_SIZE:_Q_SIZE + _KV_SIZE, _R:2 * _R].set(lora_B_k[0])
    b_exp = b_exp.at[_Q_SIZE + _KV_SIZE:, 2 * _R:].set(lora_B_v[0])
    a_stack = lora_A[0].reshape(3 * _R, _HIDDEN)

    out = pl.pallas_call(
        _body,
        grid=(m_total // _BM,),
        in_specs=[
            pl.BlockSpec((_BM, _HIDDEN), lambda m: (m, 0)),
            pl.BlockSpec((_OUT_SIZE, _HIDDEN), lambda m: (0, 0)),
            pl.BlockSpec((_OUT_SIZE, 3 * _R), lambda m: (0, 0)),
            pl.BlockSpec((3 * _R, _HIDDEN), lambda m: (0, 0)),
        ],
        out_specs=pl.BlockSpec((_BM, _OUT_SIZE), lambda m: (m, 0)),
        out_shape=jax.ShapeDtypeStruct((m_total, _OUT_SIZE), jnp.float32),
        scratch_shapes=[pltpu.VMEM((_OUT_SIZE, _HIDDEN), jnp.bfloat16)],
        compiler_params=pltpu.CompilerParams(
            dimension_semantics=("arbitrary",)),
    )(x_flat, weight, b_exp, a_stack)
    return out.reshape(*orig_shape[:-1], _OUT_SIZE)
